# baseline (speedup 1.0000x reference)
import functools

import jax
import jax.numpy as jnp
from jax import lax
from jax.experimental import pallas as pl
from jax.experimental.pallas import tpu as pltpu
from jax.experimental.pallas import tpu_sc as plsc

_TOPK = 20
_CAP = 1.0 - 1e-07
_L = 16
_NW = 32


def _sc_probe(x_flat, rows, cols):
    rows_per_w = rows // _NW
    mesh = plsc.VectorSubcoreMesh(core_axis_name="c", subcore_axis_name="s")

    @functools.partial(
        pl.kernel,
        out_type=jax.ShapeDtypeStruct((_NW * _L,), jnp.float32),
        mesh=mesh,
        scratch_types=[
            pltpu.VMEM((rows_per_w * cols,), jnp.float32),
        ],
        compiler_params=pltpu.CompilerParams(needs_layout_passes=False),
    )
    def sc_kernel(x_hbm, out_hbm, xl):
        wid = lax.axis_index("s") * 2 + lax.axis_index("c")
        pltpu.sync_copy(x_hbm.at[pl.ds(wid * rows_per_w * cols, rows_per_w * cols)], xl)
        v = jnp.maximum(xl[pl.ds(0, _L)], xl[pl.ds(_L, _L)])
        xl[pl.ds(0, _L)] = v
        pltpu.sync_copy(xl.at[pl.ds(0, _L)], out_hbm.at[pl.ds(wid * _L, _L)])

    return sc_kernel(x_flat)


def _noisy_or_body(x_ref, lt_ref, o_ref):
    x = x_ref[...]
    xb = jax.lax.bitcast_convert_type(x, jnp.int32)
    rows = x.shape[0]

    def step(i, cand):
        bit = 30 - i
        trial = cand | (1 << bit)
        cnt = jnp.sum((xb >= trial).astype(jnp.int32), axis=1, keepdims=True)
        return jnp.where(cnt >= _TOPK, trial, cand)

    cand0 = jnp.zeros((rows, 1), jnp.int32)
    tb = jax.lax.fori_loop(0, 31, step, cand0)
    tf = jax.lax.bitcast_convert_type(tb, jnp.float32)
    inv_t = jnp.exp(-lt_ref[0])

    def log_survival(v):
        scaled = jnp.exp(jnp.log(v) * inv_t)
        return jnp.log1p(-jnp.minimum(scaled, _CAP))

    strict = xb > tb
    cnt_strict = jnp.sum(strict.astype(jnp.int32), axis=1, keepdims=True)
    s = jnp.sum(jnp.where(strict, log_survival(x), 0.0), axis=1, keepdims=True)
    s = s + (_TOPK - cnt_strict).astype(jnp.float32) * log_survival(tf)
    o_ref[...] = 1.0 - jnp.exp(s)


def _tc_kernel(site_probs, lt):
    return pl.pallas_call(
        _noisy_or_body,
        out_shape=jax.ShapeDtypeStruct((site_probs.shape[0], 1), jnp.float32),
        in_specs=[
            pl.BlockSpec(memory_space=pltpu.VMEM),
            pl.BlockSpec(memory_space=pltpu.SMEM),
        ],
        out_specs=pl.BlockSpec(memory_space=pltpu.VMEM),
    )(site_probs, lt)


def kernel(site_probs, log_temperature):
    rows, cols = site_probs.shape
    lt = jnp.reshape(log_temperature, (1,)).astype(jnp.float32)
    sc_o = _sc_probe(site_probs.reshape(-1), rows, cols)
    tc_o = _tc_kernel(site_probs, lt)
    return tc_o + sc_o[: rows].reshape(rows, 1) * 0.0
